# baseline (speedup 1.0000x reference)
import functools

import jax
import jax.numpy as jnp
from jax import lax
from jax.experimental import pallas as pl
from jax.experimental.pallas import tpu as pltpu
from jax.experimental.pallas import tpu_sc as plsc

ROWS = 8192
DIM = 4096
LANES = 16
NUM_CORES = 2
NUM_SUBCORES = 16
NW = NUM_CORES * NUM_SUBCORES
ROWS_PER_W = ROWS // NW
BLK = 4
NBLK = ROWS_PER_W // BLK
NCHUNK = DIM // LANES


def _permute_body(x_hbm, idx_hbm, out_hbm, idx_v, in_v, out_v):
    wid = lax.axis_index("s") * NUM_CORES + lax.axis_index("c")
    base = wid * ROWS_PER_W
    pltpu.sync_copy(idx_hbm, idx_v)

    def block_body(b, carry):
        elem0 = (base + b * BLK) * DIM
        pltpu.sync_copy(x_hbm.at[pl.ds(elem0, BLK * DIM)], in_v)

        def chunk_body(j, c2):
            cv = idx_v[pl.ds(j * LANES, LANES)]
            for r in range(BLK):
                vals = plsc.load_gather(in_v, [cv + (r * DIM)])
                out_v[pl.ds(r * DIM + j * LANES, LANES)] = vals
            return c2

        lax.fori_loop(0, NCHUNK, chunk_body, 0)
        pltpu.sync_copy(out_v, out_hbm.at[pl.ds(elem0, BLK * DIM)])
        return carry

    lax.fori_loop(0, NBLK, block_body, 0)


@jax.jit
def _permute(x_flat, perm):
    mesh = plsc.VectorSubcoreMesh(core_axis_name="c", subcore_axis_name="s")
    f = functools.partial(
        pl.kernel,
        mesh=mesh,
        out_type=jax.ShapeDtypeStruct((ROWS * DIM,), jnp.float32),
        scratch_types=[
            pltpu.VMEM((DIM,), jnp.int32),
            pltpu.VMEM((BLK * DIM,), jnp.float32),
            pltpu.VMEM((BLK * DIM,), jnp.float32),
        ],
        compiler_params=pltpu.CompilerParams(needs_layout_passes=False),
    )(_permute_body)
    return f(x_flat, perm)


def kernel(x, shuffled_indices, inverse_indices, reverse):
    perm = jnp.where(jnp.asarray(reverse), inverse_indices, shuffled_indices)
    y = _permute(x.reshape(-1), perm).reshape(ROWS, DIM)
    objective = jnp.zeros((), dtype=jnp.float32)
    return (y, objective)


# speedup vs baseline: 1.2051x; 1.2051x over previous
import functools

import jax
import jax.numpy as jnp
from jax import lax
from jax.experimental import pallas as pl
from jax.experimental.pallas import tpu as pltpu
from jax.experimental.pallas import tpu_sc as plsc

ROWS = 8192
DIM = 4096
LANES = 16
NUM_CORES = 2
NUM_SUBCORES = 16
NW = NUM_CORES * NUM_SUBCORES
ROWS_PER_W = ROWS // NW
BLK = 4
NBLK = ROWS_PER_W // BLK
NS = NBLK // 2
NCHUNK = DIM // LANES
U = 4


def _permute_body(x_hbm, idx_hbm, out_hbm, idx_v,
                  in_a, in_b, out_a, out_b, si_a, si_b, so_a, so_b):
    wid = lax.axis_index("s") * NUM_CORES + lax.axis_index("c")
    base = wid * ROWS_PER_W
    pltpu.sync_copy(idx_hbm, idx_v)

    def in_slice(b):
        return x_hbm.at[pl.ds((base + b * BLK) * DIM, BLK * DIM)]

    def out_slice(b):
        return out_hbm.at[pl.ds((base + b * BLK) * DIM, BLK * DIM)]

    def gather_block(in_ref, out_ref):
        def chunk_body(jj, c):
            for u in range(U):
                j = jj * U + u
                cv = idx_v[pl.ds(j * LANES, LANES)]
                for r in range(BLK):
                    vals = plsc.load_gather(in_ref, [cv + (r * DIM)])
                    out_ref[pl.ds(r * DIM + j * LANES, LANES)] = vals
            return c

        lax.fori_loop(0, NCHUNK // U, chunk_body, 0)

    pltpu.async_copy(in_slice(0), in_a, si_a)
    pltpu.async_copy(in_slice(1), in_b, si_b)

    pltpu.make_async_copy(in_slice(0), in_a, si_a).wait()
    gather_block(in_a, out_a)
    pltpu.async_copy(out_a, out_slice(0), so_a)
    pltpu.async_copy(in_slice(2), in_a, si_a)
    pltpu.make_async_copy(in_slice(0), in_b, si_b).wait()
    gather_block(in_b, out_b)
    pltpu.async_copy(out_b, out_slice(1), so_b)
    pltpu.async_copy(in_slice(3), in_b, si_b)

    def super_body(s, c):
        b0 = 2 * s
        pltpu.make_async_copy(in_slice(0), in_a, si_a).wait()
        pltpu.make_async_copy(out_a, out_slice(0), so_a).wait()
        gather_block(in_a, out_a)
        pltpu.async_copy(out_a, out_slice(b0), so_a)

        @pl.when(s < NS - 1)
        def _():
            pltpu.async_copy(in_slice(b0 + 2), in_a, si_a)

        pltpu.make_async_copy(in_slice(0), in_b, si_b).wait()
        pltpu.make_async_copy(out_b, out_slice(0), so_b).wait()
        gather_block(in_b, out_b)
        pltpu.async_copy(out_b, out_slice(b0 + 1), so_b)

        @pl.when(s < NS - 1)
        def _():
            pltpu.async_copy(in_slice(b0 + 3), in_b, si_b)

        return c

    lax.fori_loop(1, NS, super_body, 0)

    pltpu.make_async_copy(out_a, out_slice(0), so_a).wait()
    pltpu.make_async_copy(out_b, out_slice(0), so_b).wait()


@jax.jit
def _permute(x_flat, perm):
    mesh = plsc.VectorSubcoreMesh(core_axis_name="c", subcore_axis_name="s")
    f = functools.partial(
        pl.kernel,
        mesh=mesh,
        out_type=jax.ShapeDtypeStruct((ROWS * DIM,), jnp.float32),
        scratch_types=[
            pltpu.VMEM((DIM,), jnp.int32),
            pltpu.VMEM((BLK * DIM,), jnp.float32),
            pltpu.VMEM((BLK * DIM,), jnp.float32),
            pltpu.VMEM((BLK * DIM,), jnp.float32),
            pltpu.VMEM((BLK * DIM,), jnp.float32),
            pltpu.SemaphoreType.DMA,
            pltpu.SemaphoreType.DMA,
            pltpu.SemaphoreType.DMA,
            pltpu.SemaphoreType.DMA,
        ],
        compiler_params=pltpu.CompilerParams(needs_layout_passes=False),
    )(_permute_body)
    return f(x_flat, perm)


def kernel(x, shuffled_indices, inverse_indices, reverse):
    perm = jnp.where(jnp.asarray(reverse), inverse_indices, shuffled_indices)
    y = _permute(x.reshape(-1), perm).reshape(ROWS, DIM)
    objective = jnp.zeros((), dtype=jnp.float32)
    return (y, objective)


# speedup vs baseline: 2.1079x; 1.7491x over previous
import functools

import jax
import jax.numpy as jnp
from jax import lax
from jax.experimental import pallas as pl
from jax.experimental.pallas import tpu as pltpu
from jax.experimental.pallas import tpu_sc as plsc

ROWS = 8192
DIM = 4096
LANES = 16
NUM_CORES = 2
NUM_SUBCORES = 16
NW = NUM_CORES * NUM_SUBCORES
ROWS_PER_W = ROWS // NW
BLK = 4
NBLK = ROWS_PER_W // BLK
NS = NBLK // 2
NCHUNK = DIM // LANES
U = 4


def _permute_body(x_hbm, idx_hbm, out_hbm, idx_v,
                  in_a, in_b, out_a, out_b, si_a, si_b, so_a, so_b):
    wid = lax.axis_index("s") * NUM_CORES + lax.axis_index("c")
    base = wid * ROWS_PER_W
    pltpu.sync_copy(idx_hbm, idx_v)

    def in_slice(b):
        return x_hbm.at[pl.ds((base + b * BLK) * DIM, BLK * DIM)]

    def out_slice(b):
        return out_hbm.at[pl.ds((base + b * BLK) * DIM, BLK * DIM)]

    def gather_block(in_ref, out_ref):
        @plsc.parallel_loop(0, NCHUNK, step=1, unroll=U)
        def _(j):
            cv = idx_v[pl.ds(j * LANES, LANES)]
            for r in range(BLK):
                vals = plsc.load_gather(in_ref, [cv + (r * DIM)])
                out_ref[pl.ds(r * DIM + j * LANES, LANES)] = vals

    pltpu.async_copy(in_slice(0), in_a, si_a)
    pltpu.async_copy(in_slice(1), in_b, si_b)

    pltpu.make_async_copy(in_slice(0), in_a, si_a).wait()
    gather_block(in_a, out_a)
    pltpu.async_copy(out_a, out_slice(0), so_a)
    pltpu.async_copy(in_slice(2), in_a, si_a)
    pltpu.make_async_copy(in_slice(0), in_b, si_b).wait()
    gather_block(in_b, out_b)
    pltpu.async_copy(out_b, out_slice(1), so_b)
    pltpu.async_copy(in_slice(3), in_b, si_b)

    def super_body(s, c):
        b0 = 2 * s
        pltpu.make_async_copy(in_slice(0), in_a, si_a).wait()
        pltpu.make_async_copy(out_a, out_slice(0), so_a).wait()
        gather_block(in_a, out_a)
        pltpu.async_copy(out_a, out_slice(b0), so_a)

        @pl.when(s < NS - 1)
        def _():
            pltpu.async_copy(in_slice(b0 + 2), in_a, si_a)

        pltpu.make_async_copy(in_slice(0), in_b, si_b).wait()
        pltpu.make_async_copy(out_b, out_slice(0), so_b).wait()
        gather_block(in_b, out_b)
        pltpu.async_copy(out_b, out_slice(b0 + 1), so_b)

        @pl.when(s < NS - 1)
        def _():
            pltpu.async_copy(in_slice(b0 + 3), in_b, si_b)

        return c

    lax.fori_loop(1, NS, super_body, 0)

    pltpu.make_async_copy(out_a, out_slice(0), so_a).wait()
    pltpu.make_async_copy(out_b, out_slice(0), so_b).wait()


@jax.jit
def _permute(x_flat, perm):
    mesh = plsc.VectorSubcoreMesh(core_axis_name="c", subcore_axis_name="s")
    f = functools.partial(
        pl.kernel,
        mesh=mesh,
        out_type=jax.ShapeDtypeStruct((ROWS * DIM,), jnp.float32),
        scratch_types=[
            pltpu.VMEM((DIM,), jnp.int32),
            pltpu.VMEM((BLK * DIM,), jnp.float32),
            pltpu.VMEM((BLK * DIM,), jnp.float32),
            pltpu.VMEM((BLK * DIM,), jnp.float32),
            pltpu.VMEM((BLK * DIM,), jnp.float32),
            pltpu.SemaphoreType.DMA,
            pltpu.SemaphoreType.DMA,
            pltpu.SemaphoreType.DMA,
            pltpu.SemaphoreType.DMA,
        ],
        compiler_params=pltpu.CompilerParams(needs_layout_passes=False),
    )(_permute_body)
    return f(x_flat, perm)


def kernel(x, shuffled_indices, inverse_indices, reverse):
    perm = jnp.where(jnp.asarray(reverse), inverse_indices, shuffled_indices)
    y = _permute(x.reshape(-1), perm).reshape(ROWS, DIM)
    objective = jnp.zeros((), dtype=jnp.float32)
    return (y, objective)


# speedup vs baseline: 3.8682x; 1.8351x over previous
import functools

import jax
import jax.numpy as jnp
from jax import lax
from jax.experimental import pallas as pl
from jax.experimental.pallas import tpu as pltpu
from jax.experimental.pallas import tpu_sc as plsc

ROWS = 8192
DIM = 4096
LANES = 16
NUM_CORES = 2
NUM_SUBCORES = 16
NW = NUM_CORES * NUM_SUBCORES
ROWS_PER_W = ROWS // NW
BLK = 8
NBLK = ROWS_PER_W // BLK
NCHUNK = DIM // LANES
U = 4


def _permute_body(x_hbm, idx_hbm, out_hbm, idx_v, in_v, out_v):
    wid = lax.axis_index("s") * NUM_CORES + lax.axis_index("c")
    base = wid * ROWS_PER_W
    pltpu.sync_copy(idx_hbm, idx_v)

    def block_body(b, carry):
        row0 = base + b * BLK
        pltpu.sync_copy(x_hbm.at[pl.ds(row0, BLK)], in_v)

        @plsc.parallel_loop(0, NCHUNK, step=1, unroll=U)
        def _(j):
            cv = idx_v[pl.ds(j * LANES, LANES)]
            for r in range(BLK):
                rv = jnp.full((LANES,), r, jnp.int32)
                vals = plsc.load_gather(in_v, [rv, cv])
                out_v[r, pl.ds(j * LANES, LANES)] = vals

        pltpu.sync_copy(out_v, out_hbm.at[pl.ds(row0, BLK)])
        return carry

    lax.fori_loop(0, NBLK, block_body, 0)


@jax.jit
def _permute(x, perm):
    mesh = plsc.VectorSubcoreMesh(core_axis_name="c", subcore_axis_name="s")
    f = functools.partial(
        pl.kernel,
        mesh=mesh,
        out_type=jax.ShapeDtypeStruct((ROWS, DIM), jnp.float32),
        scratch_types=[
            pltpu.VMEM((DIM,), jnp.int32),
            pltpu.VMEM((BLK, DIM), jnp.float32),
            pltpu.VMEM((BLK, DIM), jnp.float32),
        ],
        compiler_params=pltpu.CompilerParams(needs_layout_passes=False),
    )(_permute_body)
    return f(x, perm)


def kernel(x, shuffled_indices, inverse_indices, reverse):
    perm = jnp.where(jnp.asarray(reverse), inverse_indices, shuffled_indices)
    y = _permute(x, perm)
    objective = jnp.zeros((), dtype=jnp.float32)
    return (y, objective)


# speedup vs baseline: 6.1872x; 1.5995x over previous
import functools

import jax
import jax.numpy as jnp
from jax import lax
from jax.experimental import pallas as pl
from jax.experimental.pallas import tpu as pltpu
from jax.experimental.pallas import tpu_sc as plsc

ROWS = 8192
DIM = 4096
LANES = 16
NUM_CORES = 2
NUM_SUBCORES = 16
NW = NUM_CORES * NUM_SUBCORES
ROWS_PER_W = ROWS // NW
BLK = 8
NBLK = ROWS_PER_W // BLK
NS = NBLK // 2
NCHUNK = DIM // LANES
HALF = DIM // 2
NHCHUNK = NCHUNK // 2
U = 4


def _permute_body(x_hbm, idx_hbm, out_hbm, idx_v,
                  in_a, in_b, out_h0, out_h1, si_a, si_b, so_h0, so_h1):
    wid = lax.axis_index("s") * NUM_CORES + lax.axis_index("c")
    base = wid * ROWS_PER_W
    pltpu.sync_copy(idx_hbm, idx_v)

    def in_slice(b):
        return x_hbm.at[pl.ds(base + b * BLK, BLK)]

    def out_slice(b, half):
        return out_hbm.at[pl.ds(base + b * BLK, BLK), pl.ds(half * HALF, HALF)]

    def gather_half(in_ref, out_ref, half):
        @plsc.parallel_loop(0, NHCHUNK, step=1, unroll=U)
        def _(j):
            cv = idx_v[pl.ds((half * NHCHUNK + j) * LANES, LANES)]
            for r in range(BLK):
                rv = jnp.full((LANES,), r, jnp.int32)
                vals = plsc.load_gather(in_ref, [rv, cv])
                out_ref[r, pl.ds(j * LANES, LANES)] = vals

    def wait_in(buf, sem):
        pltpu.make_async_copy(in_slice(0), buf, sem).wait()

    def wait_out(buf, half, sem):
        pltpu.make_async_copy(buf, out_slice(0, half), sem).wait()

    def do_block(b, in_buf, in_sem, first):
        if not first:
            wait_out(out_h0, 0, so_h0)
        gather_half(in_buf, out_h0, 0)
        pltpu.async_copy(out_h0, out_slice(b, 0), so_h0)
        if not first:
            wait_out(out_h1, 1, so_h1)
        gather_half(in_buf, out_h1, 1)
        pltpu.async_copy(out_h1, out_slice(b, 1), so_h1)

    pltpu.async_copy(in_slice(0), in_a, si_a)

    wait_in(in_a, si_a)
    pltpu.async_copy(in_slice(1), in_b, si_b)
    do_block(0, in_a, si_a, first=True)
    wait_in(in_b, si_b)
    pltpu.async_copy(in_slice(2), in_a, si_a)
    do_block(1, in_b, si_b, first=False)

    def super_body(s, c):
        b0 = 2 * s
        wait_in(in_a, si_a)

        @pl.when(b0 + 1 < NBLK)
        def _():
            pltpu.async_copy(in_slice(b0 + 1), in_b, si_b)

        do_block(b0, in_a, si_a, first=False)
        wait_in(in_b, si_b)

        @pl.when(b0 + 2 < NBLK)
        def _():
            pltpu.async_copy(in_slice(b0 + 2), in_a, si_a)

        do_block(b0 + 1, in_b, si_b, first=False)
        return c

    lax.fori_loop(1, NS, super_body, 0)

    pltpu.make_async_copy(out_h0, out_slice(0, 0), so_h0).wait()
    pltpu.make_async_copy(out_h1, out_slice(0, 1), so_h1).wait()


@jax.jit
def _permute(x, perm):
    mesh = plsc.VectorSubcoreMesh(core_axis_name="c", subcore_axis_name="s")
    f = functools.partial(
        pl.kernel,
        mesh=mesh,
        out_type=jax.ShapeDtypeStruct((ROWS, DIM), jnp.float32),
        scratch_types=[
            pltpu.VMEM((DIM,), jnp.int32),
            pltpu.VMEM((BLK, DIM), jnp.float32),
            pltpu.VMEM((BLK, DIM), jnp.float32),
            pltpu.VMEM((BLK, HALF), jnp.float32),
            pltpu.VMEM((BLK, HALF), jnp.float32),
            pltpu.SemaphoreType.DMA,
            pltpu.SemaphoreType.DMA,
            pltpu.SemaphoreType.DMA,
            pltpu.SemaphoreType.DMA,
        ],
        compiler_params=pltpu.CompilerParams(needs_layout_passes=False),
    )(_permute_body)
    return f(x, perm)


def kernel(x, shuffled_indices, inverse_indices, reverse):
    perm = jnp.where(jnp.asarray(reverse), inverse_indices, shuffled_indices)
    y = _permute(x, perm)
    objective = jnp.zeros((), dtype=jnp.float32)
    return (y, objective)


# speedup vs baseline: 6.1934x; 1.0010x over previous
import functools

import jax
import jax.numpy as jnp
from jax import lax
from jax.experimental import pallas as pl
from jax.experimental.pallas import tpu as pltpu
from jax.experimental.pallas import tpu_sc as plsc

ROWS = 8192
DIM = 4096
LANES = 16
NUM_CORES = 2
NUM_SUBCORES = 16
NW = NUM_CORES * NUM_SUBCORES
ROWS_PER_W = ROWS // NW
BLK = 8
NBLK = ROWS_PER_W // BLK
NS = NBLK // 2
NCHUNK = DIM // LANES
HALF = DIM // 2
NHCHUNK = NCHUNK // 2
U = 8


def _permute_body(x_hbm, idx_hbm, out_hbm, idx_v,
                  in_a, in_b, out_h0, out_h1, si_a, si_b, so_h0, so_h1):
    wid = lax.axis_index("s") * NUM_CORES + lax.axis_index("c")
    base = wid * ROWS_PER_W
    pltpu.sync_copy(idx_hbm, idx_v)

    def in_slice(b):
        return x_hbm.at[pl.ds(base + b * BLK, BLK)]

    def out_slice(b, half):
        return out_hbm.at[pl.ds(base + b * BLK, BLK), pl.ds(half * HALF, HALF)]

    def gather_half(in_ref, out_ref, half):
        @plsc.parallel_loop(0, NHCHUNK, step=1, unroll=U)
        def _(j):
            cv = idx_v[pl.ds((half * NHCHUNK + j) * LANES, LANES)]
            for r in range(BLK):
                rv = jnp.full((LANES,), r, jnp.int32)
                vals = plsc.load_gather(in_ref, [rv, cv])
                out_ref[r, pl.ds(j * LANES, LANES)] = vals

    def wait_in(buf, sem):
        pltpu.make_async_copy(in_slice(0), buf, sem).wait()

    def wait_out(buf, half, sem):
        pltpu.make_async_copy(buf, out_slice(0, half), sem).wait()

    def do_block(b, in_buf, in_sem, first):
        if not first:
            wait_out(out_h0, 0, so_h0)
        gather_half(in_buf, out_h0, 0)
        pltpu.async_copy(out_h0, out_slice(b, 0), so_h0)
        if not first:
            wait_out(out_h1, 1, so_h1)
        gather_half(in_buf, out_h1, 1)
        pltpu.async_copy(out_h1, out_slice(b, 1), so_h1)

    pltpu.async_copy(in_slice(0), in_a, si_a)

    wait_in(in_a, si_a)
    pltpu.async_copy(in_slice(1), in_b, si_b)
    do_block(0, in_a, si_a, first=True)
    wait_in(in_b, si_b)
    pltpu.async_copy(in_slice(2), in_a, si_a)
    do_block(1, in_b, si_b, first=False)

    def super_body(s, c):
        b0 = 2 * s
        wait_in(in_a, si_a)

        @pl.when(b0 + 1 < NBLK)
        def _():
            pltpu.async_copy(in_slice(b0 + 1), in_b, si_b)

        do_block(b0, in_a, si_a, first=False)
        wait_in(in_b, si_b)

        @pl.when(b0 + 2 < NBLK)
        def _():
            pltpu.async_copy(in_slice(b0 + 2), in_a, si_a)

        do_block(b0 + 1, in_b, si_b, first=False)
        return c

    lax.fori_loop(1, NS, super_body, 0)

    pltpu.make_async_copy(out_h0, out_slice(0, 0), so_h0).wait()
    pltpu.make_async_copy(out_h1, out_slice(0, 1), so_h1).wait()


@jax.jit
def _permute(x, perm):
    mesh = plsc.VectorSubcoreMesh(core_axis_name="c", subcore_axis_name="s")
    f = functools.partial(
        pl.kernel,
        mesh=mesh,
        out_type=jax.ShapeDtypeStruct((ROWS, DIM), jnp.float32),
        scratch_types=[
            pltpu.VMEM((DIM,), jnp.int32),
            pltpu.VMEM((BLK, DIM), jnp.float32),
            pltpu.VMEM((BLK, DIM), jnp.float32),
            pltpu.VMEM((BLK, HALF), jnp.float32),
            pltpu.VMEM((BLK, HALF), jnp.float32),
            pltpu.SemaphoreType.DMA,
            pltpu.SemaphoreType.DMA,
            pltpu.SemaphoreType.DMA,
            pltpu.SemaphoreType.DMA,
        ],
        compiler_params=pltpu.CompilerParams(needs_layout_passes=False),
    )(_permute_body)
    return f(x, perm)


def kernel(x, shuffled_indices, inverse_indices, reverse):
    perm = jnp.where(jnp.asarray(reverse), inverse_indices, shuffled_indices)
    y = _permute(x, perm)
    objective = jnp.zeros((), dtype=jnp.float32)
    return (y, objective)


# speedup vs baseline: 6.2589x; 1.0106x over previous
import functools

import jax
import jax.numpy as jnp
from jax import lax
from jax.experimental import pallas as pl
from jax.experimental.pallas import tpu as pltpu
from jax.experimental.pallas import tpu_sc as plsc

ROWS = 8192
DIM = 4096
LANES = 16
NUM_CORES = 2
NUM_SUBCORES = 16
NW = NUM_CORES * NUM_SUBCORES
ROWS_PER_W = ROWS // NW
BLK = 8
NBLK = ROWS_PER_W // BLK
NS = NBLK // 2
NCHUNK = DIM // LANES
HALF = DIM // 2
NHCHUNK = NCHUNK // 2
U = 8


def _permute_body(x_hbm, idx_hbm, out_hbm, idx_v,
                  in_a, in_b, out_h0, out_h1, si_a, si_b, so_h0, so_h1):
    wid = lax.axis_index("s") * NUM_CORES + lax.axis_index("c")
    base = wid * ROWS_PER_W
    pltpu.sync_copy(idx_hbm, idx_v)

    def in_slice(b):
        return x_hbm.at[pl.ds(base + b * BLK, BLK)]

    def out_slice(b, half):
        return out_hbm.at[pl.ds(base + b * BLK, BLK), pl.ds(half * HALF, HALF)]

    def gather_half(in_ref, out_ref, half):
        @plsc.parallel_loop(0, NHCHUNK, step=1, unroll=U)
        def _(j):
            cv = idx_v[pl.ds((half * NHCHUNK + j) * LANES, LANES)]
            for r in range(BLK):
                rv = jnp.full((LANES,), r, jnp.int32)
                vals = plsc.load_gather(in_ref, [rv, cv])
                out_ref[r, pl.ds(j * LANES, LANES)] = vals

    def wait_in(buf, sem):
        pltpu.make_async_copy(in_slice(0), buf, sem).wait()

    def wait_out(buf, half, sem):
        pltpu.make_async_copy(buf, out_slice(0, half), sem).wait()

    def do_block(b, in_buf, in_sem, first):
        if not first:
            wait_out(out_h0, 0, so_h0)
        gather_half(in_buf, out_h0, 0)
        pltpu.async_copy(out_h0, out_slice(b, 0), so_h0)
        if not first:
            wait_out(out_h1, 1, so_h1)
        gather_half(in_buf, out_h1, 1)
        pltpu.async_copy(out_h1, out_slice(b, 1), so_h1)

    pltpu.async_copy(in_slice(0), in_a, si_a)
    pltpu.async_copy(in_slice(1), in_b, si_b)

    wait_in(in_a, si_a)
    do_block(0, in_a, si_a, first=True)
    pltpu.async_copy(in_slice(2), in_a, si_a)
    wait_in(in_b, si_b)
    do_block(1, in_b, si_b, first=False)

    def super_body(s, c):
        b0 = 2 * s

        @pl.when(b0 + 1 < NBLK)
        def _():
            pltpu.async_copy(in_slice(b0 + 1), in_b, si_b)

        wait_in(in_a, si_a)
        do_block(b0, in_a, si_a, first=False)

        @pl.when(b0 + 2 < NBLK)
        def _():
            pltpu.async_copy(in_slice(b0 + 2), in_a, si_a)

        wait_in(in_b, si_b)
        do_block(b0 + 1, in_b, si_b, first=False)
        return c

    lax.fori_loop(1, NS, super_body, 0)

    pltpu.make_async_copy(out_h0, out_slice(0, 0), so_h0).wait()
    pltpu.make_async_copy(out_h1, out_slice(0, 1), so_h1).wait()


@jax.jit
def _permute(x, perm):
    mesh = plsc.VectorSubcoreMesh(core_axis_name="c", subcore_axis_name="s")
    f = functools.partial(
        pl.kernel,
        mesh=mesh,
        out_type=jax.ShapeDtypeStruct((ROWS, DIM), jnp.float32),
        scratch_types=[
            pltpu.VMEM((DIM,), jnp.int32),
            pltpu.VMEM((BLK, DIM), jnp.float32),
            pltpu.VMEM((BLK, DIM), jnp.float32),
            pltpu.VMEM((BLK, HALF), jnp.float32),
            pltpu.VMEM((BLK, HALF), jnp.float32),
            pltpu.SemaphoreType.DMA,
            pltpu.SemaphoreType.DMA,
            pltpu.SemaphoreType.DMA,
            pltpu.SemaphoreType.DMA,
        ],
        compiler_params=pltpu.CompilerParams(needs_layout_passes=False),
    )(_permute_body)
    return f(x, perm)


def kernel(x, shuffled_indices, inverse_indices, reverse):
    perm = jnp.where(jnp.asarray(reverse), inverse_indices, shuffled_indices)
    y = _permute(x, perm)
    objective = jnp.zeros((), dtype=jnp.float32)
    return (y, objective)
